# Initial kernel scaffold; baseline (speedup 1.0000x reference)
import jax
import jax.numpy as jnp
from jax.experimental import pallas as pl


def kernel(nf1, nf2, nf3, nf4, go_embed, go_rad, rel_embed, bn_gamma, bn_beta):
    raise NotImplementedError("write your pallas kernel here")


# speedup vs baseline: 1.5166x; 1.5166x over previous
import functools

import jax
import jax.numpy as jnp
from jax import lax
from jax.experimental import pallas as pl
from jax.experimental.pallas import tpu as pltpu
from jax.experimental.pallas import tpu_sc as plsc

D = 64
B = 16384
NMAT = 9
NIDX = NMAT * B
NW = 32
CHUNK = 128
CPW = NIDX // (NW * CHUNK)
MARGIN = 0.1
EPS = 1e-5


def _sc_gather(go_embed, go_rad_flat, idx):
    idx3d = idx.reshape(NW, CPW, CHUNK)

    @functools.partial(
        pl.kernel,
        out_type=[
            jax.ShapeDtypeStruct((NIDX, D), jnp.float32),
            jax.ShapeDtypeStruct((NIDX,), jnp.float32),
        ],
        mesh=plsc.VectorSubcoreMesh(core_axis_name="c", subcore_axis_name="s"),
        compiler_params=pltpu.CompilerParams(use_tc_tiling_on_sc=False),
        scratch_types=[
            pltpu.VMEM((CPW, CHUNK), jnp.int32),
            pltpu.VMEM((CHUNK, D), jnp.float32),
            pltpu.VMEM((CHUNK,), jnp.float32),
            pltpu.SemaphoreType.DMA,
            pltpu.SemaphoreType.DMA,
        ],
    )
    def gather_kernel(emb_hbm, rad_hbm, idx_hbm, rows_out, rad_out,
                      idx_v, rows_v, rad_v, sem_r, sem_d):
        wid = lax.axis_index("s") * 2 + lax.axis_index("c")
        pltpu.sync_copy(idx_hbm.at[wid], idx_v)

        def body(j, carry):
            g_rows = pltpu.async_copy(emb_hbm.at[idx_v.at[j]], rows_v, sem_r)
            g_rad = pltpu.async_copy(rad_hbm.at[idx_v.at[j]], rad_v, sem_d)
            g_rows.wait()
            g_rad.wait()
            base = (wid * CPW + j) * CHUNK
            pltpu.sync_copy(rows_v, rows_out.at[pl.ds(base, CHUNK)])
            pltpu.sync_copy(rad_v, rad_out.at[pl.ds(base, CHUNK)])
            return carry

        lax.fori_loop(0, CPW, body, 0)

    return gather_kernel(go_embed, go_rad_flat, idx3d)


NBLK = 8
R = B // NBLK


def _stats_kernel(rows_ref, out_ref):
    x = rows_ref[:, :]
    s = jnp.sum(x, axis=0, keepdims=True)
    q = jnp.sum(x * x, axis=0, keepdims=True)
    out_ref[0, :, :] = jnp.concatenate([s, q], axis=1)


def _relu(x):
    return jnp.maximum(x, 0.0)


def _loss_kernel(rows0, rows1, rows2, rows3, rows4, rows5, rows6, rows7,
                 rows8, stats_ref, rads_ref, ri3_ref, ri4_ref, rel_ref,
                 rad16_ref, g_ref, b_ref, out_ref):
    i = pl.program_id(0)
    mats = (rows0, rows1, rows2, rows3, rows4, rows5, rows6, rows7, rows8)
    gamma, beta = g_ref[:, :], b_ref[:, :]
    stats = stats_ref[:, :]

    def scale_shift(k):
        s = stats[k:k + 1, 0:D]
        q = stats[k:k + 1, D:2 * D]
        m = s * (1.0 / B)
        v = q * (1.0 / B) - m * m
        scale = lax.rsqrt(v + EPS) * gamma
        shift = beta - m * scale
        return scale, shift

    def bnx(k):
        scale, shift = scale_shift(k)
        return mats[k][:, :] * scale + shift

    def rownorm(x):
        return jnp.sqrt(jnp.sum(x * x, axis=1))

    def radv(k):
        return jnp.abs(rads_ref[k, :])

    part = jnp.sum(_relu(rownorm(bnx(0) - bnx(1)) + radv(0) - radv(1)
                         - MARGIN))
    c = bnx(2)
    d = bnx(3)
    e = bnx(4)
    rc = radv(2)
    rd = radv(3)
    part += jnp.sum(_relu(rownorm(c - d) - (rc + rd) - MARGIN)
                    + _relu(rownorm(e - c) - rc - MARGIN)
                    + _relu(rownorm(e - d) - rd - MARGIN))
    iota = lax.broadcasted_iota(jnp.int32, (R, 16), 1)
    oh3 = (iota == ri3_ref[:, :]).astype(jnp.float32)
    oh4 = (iota == ri4_ref[:, :]).astype(jnp.float32)
    rE3 = jnp.dot(oh3, rel_ref[:, :], preferred_element_type=jnp.float32)
    rE4 = jnp.dot(oh4, rel_ref[:, :], preferred_element_type=jnp.float32)
    rc4 = jnp.dot(oh4, jnp.abs(rad16_ref[:, :]),
                  preferred_element_type=jnp.float32)[:, 0]
    part += jnp.sum(_relu(rownorm(bnx(5) + rE3 - bnx(6)) + radv(5) - radv(6)
                          - MARGIN))
    part += jnp.sum(_relu(rownorm(bnx(7) - (bnx(8) + rE4)) - (rc4 + radv(8))
                          - MARGIN))

    @pl.when(i == 0)
    def _():
        out_ref[:, :] = jnp.zeros((1, 1), jnp.float32)

    out_ref[:, :] += jnp.reshape(part * (1.0 / B), (1, 1))


def _tc_loss(rows, rads, ri3, ri4, rel_embed, rad16, gamma, beta):
    stats = pl.pallas_call(
        _stats_kernel,
        grid=(NMAT,),
        out_shape=jax.ShapeDtypeStruct((NMAT, 1, 2 * D), jnp.float32),
        in_specs=[pl.BlockSpec((B, D), lambda k: (k, 0))],
        out_specs=pl.BlockSpec((1, 1, 2 * D), lambda k: (k, 0, 0)),
    )(rows)
    stats = stats.reshape(NMAT, 2 * D)

    def mat_spec(k):
        return pl.BlockSpec((R, D), lambda i, k=k: (k * NBLK + i, 0))

    def full_spec(shape):
        nd = len(shape)
        return pl.BlockSpec(shape, lambda i: (0,) * nd)

    loss = pl.pallas_call(
        _loss_kernel,
        grid=(NBLK,),
        out_shape=jax.ShapeDtypeStruct((1, 1), jnp.float32),
        in_specs=[mat_spec(k) for k in range(NMAT)] + [
            full_spec(stats.shape),
            pl.BlockSpec((NMAT, R), lambda i: (0, i)),
            pl.BlockSpec((R, 1), lambda i: (i, 0)),
            pl.BlockSpec((R, 1), lambda i: (i, 0)),
            full_spec(rel_embed.shape),
            full_spec(rad16.shape),
            full_spec(gamma.shape),
            full_spec(beta.shape),
        ],
        out_specs=full_spec((1, 1)),
    )(rows, rows, rows, rows, rows, rows, rows, rows, rows,
      stats, rads, ri3, ri4, rel_embed, rad16, gamma, beta)
    return loss


def kernel(nf1, nf2, nf3, nf4, go_embed, go_rad, rel_embed, bn_gamma, bn_beta):
    nf1 = nf1.astype(jnp.int32)
    nf2 = nf2.astype(jnp.int32)
    nf3 = nf3.astype(jnp.int32)
    nf4 = nf4.astype(jnp.int32)
    idx = jnp.concatenate([
        nf1[:, 0], nf1[:, 1],
        nf2[:, 0], nf2[:, 1], nf2[:, 2],
        nf3[:, 1], nf3[:, 2],
        nf4[:, 0], nf4[:, 2],
    ])
    rows, rad_g = _sc_gather(go_embed, go_rad.reshape(-1), idx)
    rads = rad_g.reshape(NMAT, B)
    loss = _tc_loss(rows, rads,
                    nf3[:, 0:1], nf4[:, 1:2],
                    rel_embed, go_rad[:16],
                    bn_gamma.reshape(1, D), bn_beta.reshape(1, D))
    return loss[0, 0]


# speedup vs baseline: 1.9015x; 1.2538x over previous
import functools

import jax
import jax.numpy as jnp
from jax import lax
from jax.experimental import pallas as pl
from jax.experimental.pallas import tpu as pltpu
from jax.experimental.pallas import tpu_sc as plsc

D = 64
B = 16384
NMAT = 9
NIDX = NMAT * B
NW = 32
CHUNK = 128
CPW = NIDX // (NW * CHUNK)
MARGIN = 0.1
EPS = 1e-5


def _sc_gather(go_embed, go_rad_flat, idx):
    idx3d = idx.reshape(NW, CPW, CHUNK)

    @functools.partial(
        pl.kernel,
        out_type=[
            jax.ShapeDtypeStruct((NIDX, D), jnp.float32),
            jax.ShapeDtypeStruct((NIDX,), jnp.float32),
        ],
        mesh=plsc.VectorSubcoreMesh(core_axis_name="c", subcore_axis_name="s"),
        compiler_params=pltpu.CompilerParams(use_tc_tiling_on_sc=False),
        scratch_types=[
            pltpu.VMEM((CPW, CHUNK), jnp.int32),
            pltpu.VMEM((CHUNK, D), jnp.float32),
            pltpu.VMEM((CHUNK,), jnp.float32),
            pltpu.SemaphoreType.DMA,
            pltpu.SemaphoreType.DMA,
        ],
    )
    def gather_kernel(emb_hbm, rad_hbm, idx_hbm, rows_out, rad_out,
                      idx_v, rows_v, rad_v, sem_r, sem_d):
        wid = lax.axis_index("s") * 2 + lax.axis_index("c")
        pltpu.sync_copy(idx_hbm.at[wid], idx_v)

        def body(j, carry):
            g_rows = pltpu.async_copy(emb_hbm.at[idx_v.at[j]], rows_v, sem_r)
            g_rad = pltpu.async_copy(rad_hbm.at[idx_v.at[j]], rad_v, sem_d)
            g_rows.wait()
            g_rad.wait()
            base = (wid * CPW + j) * CHUNK
            pltpu.sync_copy(rows_v, rows_out.at[pl.ds(base, CHUNK)])
            pltpu.sync_copy(rad_v, rad_out.at[pl.ds(base, CHUNK)])
            return carry

        lax.fori_loop(0, CPW, body, 0)

    return gather_kernel(go_embed, go_rad_flat, idx3d)


PB = B // 2
NBLK = 8
RP = PB // NBLK


def _stats_kernel(p_ref, g_ref, b_ref, out_ref):
    x = p_ref[:, :]
    s = jnp.sum(x, axis=0, keepdims=True)
    q = jnp.sum(x * x, axis=0, keepdims=True)
    sd = s[:, 0:D] + s[:, D:2 * D]
    qd = q[:, 0:D] + q[:, D:2 * D]
    m = sd * (1.0 / B)
    v = qd * (1.0 / B) - m * m
    sc = lax.rsqrt(v + EPS) * g_ref[:, :]
    sh = b_ref[:, :] - m * sc
    sc128 = jnp.concatenate([sc, sc], axis=1)
    sh128 = jnp.concatenate([sh, sh], axis=1)
    out_ref[0, :, :] = jnp.concatenate([sc128, sh128], axis=1)


def _relu(x):
    return jnp.maximum(x, 0.0)


def _loss_kernel(m0, m1, m2, m3, m4, m5, m6, m7, m8, ss_ref, rads_ref,
                 ri3_ref, ri4_ref, rel_ref, rad16_ref, out_ref):
    i = pl.program_id(0)
    mats = (m0, m1, m2, m3, m4, m5, m6, m7, m8)

    def bnx(k):
        scsh = ss_ref[k:k + 1, :]
        return mats[k][:, :] * scsh[:, 0:128] + scsh[:, 128:256]

    ii = lax.broadcasted_iota(jnp.int32, (128, 2), 0)
    jj = lax.broadcasted_iota(jnp.int32, (128, 2), 1)
    half_w = jnp.where((ii < D) == (jj == 0), 1.0, 0.0).astype(jnp.float32)

    def rnorm(dd):
        return jnp.sqrt(jnp.dot(dd * dd, half_w,
                                preferred_element_type=jnp.float32))

    def radp(k):
        return jnp.abs(rads_ref[k, :, :])

    part = jnp.sum(_relu(rnorm(bnx(0) - bnx(1)) + radp(0) - radp(1)
                         - MARGIN))
    c = bnx(2)
    d = bnx(3)
    e = bnx(4)
    rc = radp(2)
    rd = radp(3)
    part += jnp.sum(_relu(rnorm(c - d) - (rc + rd) - MARGIN)
                    + _relu(rnorm(e - c) - rc - MARGIN)
                    + _relu(rnorm(e - d) - rd - MARGIN))

    i16 = lax.broadcasted_iota(jnp.int32, (RP, 16), 1)
    f32 = jnp.float32
    oh3 = jnp.concatenate([(i16 == ri3_ref[:, 0:1]).astype(f32),
                           (i16 == ri3_ref[:, 1:2]).astype(f32)], axis=1)
    oh4 = jnp.concatenate([(i16 == ri4_ref[:, 0:1]).astype(f32),
                           (i16 == ri4_ref[:, 1:2]).astype(f32)], axis=1)
    z = jnp.zeros((16, D), f32)
    relblk = jnp.concatenate(
        [jnp.concatenate([rel_ref[:, :], z], axis=1),
         jnp.concatenate([z, rel_ref[:, :]], axis=1)], axis=0)
    rE3 = jnp.dot(oh3, relblk, preferred_element_type=f32)
    rE4 = jnp.dot(oh4, relblk, preferred_element_type=f32)
    a16 = jnp.abs(rad16_ref[:, :])
    z16 = jnp.zeros((16, 1), f32)
    w4 = jnp.concatenate([jnp.concatenate([a16, z16], axis=0),
                          jnp.concatenate([z16, a16], axis=0)], axis=1)
    rc4 = jnp.dot(oh4, w4, preferred_element_type=f32)

    part += jnp.sum(_relu(rnorm(bnx(5) + rE3 - bnx(6)) + radp(5) - radp(6)
                          - MARGIN))
    part += jnp.sum(_relu(rnorm(bnx(7) - (bnx(8) + rE4)) - (rc4 + radp(8))
                          - MARGIN))

    @pl.when(i == 0)
    def _():
        out_ref[:, :] = jnp.zeros((1, 1), jnp.float32)

    out_ref[:, :] += jnp.reshape(part * (1.0 / B), (1, 1))


def _tc_loss(packed, radsp, ri3p, ri4p, rel_embed, rad16, gamma, beta):
    scsh = pl.pallas_call(
        _stats_kernel,
        grid=(NMAT,),
        out_shape=jax.ShapeDtypeStruct((NMAT, 1, 4 * D), jnp.float32),
        in_specs=[pl.BlockSpec((PB, 2 * D), lambda k: (k, 0)),
                  pl.BlockSpec((1, D), lambda k: (0, 0)),
                  pl.BlockSpec((1, D), lambda k: (0, 0))],
        out_specs=pl.BlockSpec((1, 1, 4 * D), lambda k: (k, 0, 0)),
    )(packed, gamma, beta)
    scsh = scsh.reshape(NMAT, 4 * D)

    def mat_spec(k):
        return pl.BlockSpec((RP, 2 * D), lambda i, k=k: (k * NBLK + i, 0))

    def full_spec(shape):
        nd = len(shape)
        return pl.BlockSpec(shape, lambda i: (0,) * nd)

    loss = pl.pallas_call(
        _loss_kernel,
        grid=(NBLK,),
        out_shape=jax.ShapeDtypeStruct((1, 1), jnp.float32),
        in_specs=[mat_spec(k) for k in range(NMAT)] + [
            full_spec(scsh.shape),
            pl.BlockSpec((NMAT, RP, 2), lambda i: (0, i, 0)),
            pl.BlockSpec((RP, 2), lambda i: (i, 0)),
            pl.BlockSpec((RP, 2), lambda i: (i, 0)),
            full_spec(rel_embed.shape),
            full_spec(rad16.shape),
        ],
        out_specs=full_spec((1, 1)),
    )(packed, packed, packed, packed, packed, packed, packed, packed,
      packed, scsh, radsp, ri3p, ri4p, rel_embed, rad16)
    return loss


def kernel(nf1, nf2, nf3, nf4, go_embed, go_rad, rel_embed, bn_gamma, bn_beta):
    nf1 = nf1.astype(jnp.int32)
    nf2 = nf2.astype(jnp.int32)
    nf3 = nf3.astype(jnp.int32)
    nf4 = nf4.astype(jnp.int32)
    idx = jnp.concatenate([
        nf1[:, 0], nf1[:, 1],
        nf2[:, 0], nf2[:, 1], nf2[:, 2],
        nf3[:, 1], nf3[:, 2],
        nf4[:, 0], nf4[:, 2],
    ])
    rows, rad_g = _sc_gather(go_embed, go_rad.reshape(-1), idx)
    packed = rows.reshape(NMAT * PB, 2 * D)
    radsp = rad_g.reshape(NMAT, PB, 2)
    loss = _tc_loss(packed, radsp,
                    nf3[:, 0].reshape(PB, 2), nf4[:, 1].reshape(PB, 2),
                    rel_embed, go_rad[:16],
                    bn_gamma.reshape(1, D), bn_beta.reshape(1, D))
    return loss[0, 0]


# speedup vs baseline: 2.2213x; 1.1682x over previous
import functools

import jax
import jax.numpy as jnp
from jax import lax
from jax.experimental import pallas as pl
from jax.experimental.pallas import tpu as pltpu
from jax.experimental.pallas import tpu_sc as plsc

D = 64
B = 16384
NMAT = 9
NIDX = NMAT * B
NW = 32
CHUNK = 128
CPW = NIDX // (NW * CHUNK)
MARGIN = 0.1
EPS = 1e-5


PBC = B // 2


def _sc_gather(go_embed, go_rad_flat, nf1t, nf2t, nf3t, nf4t):
    @functools.partial(
        pl.kernel,
        out_type=[
            jax.ShapeDtypeStruct((NMAT * PBC, 2 * D), jnp.float32),
            jax.ShapeDtypeStruct((NIDX,), jnp.float32),
        ],
        mesh=plsc.VectorSubcoreMesh(core_axis_name="c", subcore_axis_name="s"),
        compiler_params=pltpu.CompilerParams(use_tc_tiling_on_sc=False),
        scratch_types=[
            pltpu.VMEM((CHUNK,), jnp.int32),
            pltpu.VMEM((CHUNK,), jnp.int32),
            pltpu.VMEM((CHUNK, D), jnp.float32),
            pltpu.VMEM((CHUNK, D), jnp.float32),
            pltpu.VMEM((CHUNK,), jnp.float32),
            pltpu.VMEM((CHUNK,), jnp.float32),
            pltpu.SemaphoreType.DMA,
            pltpu.SemaphoreType.DMA,
            pltpu.SemaphoreType.DMA,
            pltpu.SemaphoreType.DMA,
        ],
    )
    def gather_kernel(emb_hbm, rad_hbm, n1_hbm, n2_hbm, n3_hbm, n4_hbm,
                      rows_out, rad_out,
                      idx0, idx1, rows0, rows1, radv0, radv1,
                      sg0, sg1, sr0, sr1):
        wid = lax.axis_index("s") * 2 + lax.axis_index("c")
        mapping = ((n1_hbm, 0), (n1_hbm, 1),
                   (n2_hbm, 0), (n2_hbm, 1), (n2_hbm, 2),
                   (n3_hbm, 1), (n3_hbm, 2),
                   (n4_hbm, 0), (n4_hbm, 2))
        slots = ((idx0, rows0, radv0, sg0, sr0),
                 (idx1, rows1, radv1, sg1, sr1))
        njobs = 4 * NMAT

        def job(j):
            k, t = j // 4, j % 4
            h, tt = t // 2, t % 2
            c = wid * 2 + tt
            return k, h, c

        def issue(j, slot):
            idxb, rowsb, radb, sg, sr = slot
            k, h, c = job(j)
            ref, col = mapping[k]
            pltpu.sync_copy(
                ref.at[col, pl.ds(h * (B // 2) + c * CHUNK, CHUNK)], idxb)
            cp_r = pltpu.async_copy(emb_hbm.at[idxb], rowsb, sg)
            cp_d = pltpu.async_copy(rad_hbm.at[idxb], radb, sr)
            return cp_r, cp_d

        def drain(j, slot, cps):
            idxb, rowsb, radb, sg, sr = slot
            cps[0].wait()
            cps[1].wait()
            k, h, c = job(j)
            pltpu.sync_copy(
                rowsb,
                rows_out.at[pl.ds(k * PBC + c * CHUNK, CHUNK),
                            pl.ds(h * D, D)])
            pltpu.sync_copy(
                radb,
                rad_out.at[pl.ds(k * B + h * (B // 2) + c * CHUNK, CHUNK)])

        cps = issue(0, slots[0])
        for j in range(njobs):
            nxt = issue(j + 1, slots[(j + 1) % 2]) if j + 1 < njobs else None
            drain(j, slots[j % 2], cps)
            cps = nxt

    return gather_kernel(go_embed, go_rad_flat, nf1t, nf2t, nf3t, nf4t)


PB = B // 2
NBLK = 8
RP = PB // NBLK


def _stats_kernel(p_ref, g_ref, b_ref, out_ref):
    x = p_ref[:, :]
    s = jnp.sum(x, axis=0, keepdims=True)
    q = jnp.sum(x * x, axis=0, keepdims=True)
    sd = s[:, 0:D] + s[:, D:2 * D]
    qd = q[:, 0:D] + q[:, D:2 * D]
    m = sd * (1.0 / B)
    v = qd * (1.0 / B) - m * m
    sc = lax.rsqrt(v + EPS) * g_ref[:, :]
    sh = b_ref[:, :] - m * sc
    sc128 = jnp.concatenate([sc, sc], axis=1)
    sh128 = jnp.concatenate([sh, sh], axis=1)
    out_ref[0, :, :] = jnp.concatenate([sc128, sh128], axis=1)


def _relu(x):
    return jnp.maximum(x, 0.0)


def _loss_kernel(m0, m1, m2, m3, m4, m5, m6, m7, m8, ss_ref, rads_ref,
                 ri3_ref, ri4_ref, rel_ref, rad16_ref, out_ref):
    i = pl.program_id(0)
    mats = (m0, m1, m2, m3, m4, m5, m6, m7, m8)

    def bnx(k):
        scsh = ss_ref[k:k + 1, :]
        return mats[k][:, :] * scsh[:, 0:128] + scsh[:, 128:256]

    ii = lax.broadcasted_iota(jnp.int32, (128, 2), 0)
    jj = lax.broadcasted_iota(jnp.int32, (128, 2), 1)
    half_w = jnp.where((ii < D) == (jj == 0), 1.0, 0.0).astype(jnp.float32)

    def rnorm(dd):
        return jnp.sqrt(jnp.dot(dd * dd, half_w,
                                preferred_element_type=jnp.float32))

    def radp(k):
        return jnp.abs(rads_ref[k, :, :])

    part = jnp.sum(_relu(rnorm(bnx(0) - bnx(1)) + radp(0) - radp(1)
                         - MARGIN))
    c = bnx(2)
    d = bnx(3)
    e = bnx(4)
    rc = radp(2)
    rd = radp(3)
    part += jnp.sum(_relu(rnorm(c - d) - (rc + rd) - MARGIN)
                    + _relu(rnorm(e - c) - rc - MARGIN)
                    + _relu(rnorm(e - d) - rd - MARGIN))

    i16 = lax.broadcasted_iota(jnp.int32, (RP, 16), 1)
    f32 = jnp.float32
    oh3 = jnp.concatenate([(i16 == ri3_ref[:, 0:1]).astype(f32),
                           (i16 == ri3_ref[:, 1:2]).astype(f32)], axis=1)
    oh4 = jnp.concatenate([(i16 == ri4_ref[:, 0:1]).astype(f32),
                           (i16 == ri4_ref[:, 1:2]).astype(f32)], axis=1)
    z = jnp.zeros((16, D), f32)
    relblk = jnp.concatenate(
        [jnp.concatenate([rel_ref[:, :], z], axis=1),
         jnp.concatenate([z, rel_ref[:, :]], axis=1)], axis=0)
    rE3 = jnp.dot(oh3, relblk, preferred_element_type=f32)
    rE4 = jnp.dot(oh4, relblk, preferred_element_type=f32)
    a16 = jnp.abs(rad16_ref[:, :])
    z16 = jnp.zeros((16, 1), f32)
    w4 = jnp.concatenate([jnp.concatenate([a16, z16], axis=0),
                          jnp.concatenate([z16, a16], axis=0)], axis=1)
    rc4 = jnp.dot(oh4, w4, preferred_element_type=f32)

    part += jnp.sum(_relu(rnorm(bnx(5) + rE3 - bnx(6)) + radp(5) - radp(6)
                          - MARGIN))
    part += jnp.sum(_relu(rnorm(bnx(7) - (bnx(8) + rE4)) - (rc4 + radp(8))
                          - MARGIN))

    @pl.when(i == 0)
    def _():
        out_ref[:, :] = jnp.zeros((1, 1), jnp.float32)

    out_ref[:, :] += jnp.reshape(part * (1.0 / B), (1, 1))


def _tc_loss(packed, radsp, ri3p, ri4p, rel_embed, rad16, gamma, beta):
    scsh = pl.pallas_call(
        _stats_kernel,
        grid=(NMAT,),
        out_shape=jax.ShapeDtypeStruct((NMAT, 1, 4 * D), jnp.float32),
        in_specs=[pl.BlockSpec((PB, 2 * D), lambda k: (k, 0)),
                  pl.BlockSpec((1, D), lambda k: (0, 0)),
                  pl.BlockSpec((1, D), lambda k: (0, 0))],
        out_specs=pl.BlockSpec((1, 1, 4 * D), lambda k: (k, 0, 0)),
    )(packed, gamma, beta)
    scsh = scsh.reshape(NMAT, 4 * D)

    def mat_spec(k):
        return pl.BlockSpec((RP, 2 * D), lambda i, k=k: (k * NBLK + i, 0))

    def full_spec(shape):
        nd = len(shape)
        return pl.BlockSpec(shape, lambda i: (0,) * nd)

    loss = pl.pallas_call(
        _loss_kernel,
        grid=(NBLK,),
        out_shape=jax.ShapeDtypeStruct((1, 1), jnp.float32),
        in_specs=[mat_spec(k) for k in range(NMAT)] + [
            full_spec(scsh.shape),
            pl.BlockSpec((NMAT, RP, 2), lambda i: (0, i, 0)),
            pl.BlockSpec((RP, 2), lambda i: (i, 0)),
            pl.BlockSpec((RP, 2), lambda i: (i, 0)),
            full_spec(rel_embed.shape),
            full_spec(rad16.shape),
        ],
        out_specs=full_spec((1, 1)),
    )(packed, packed, packed, packed, packed, packed, packed, packed,
      packed, scsh, radsp, ri3p, ri4p, rel_embed, rad16)
    return loss


def kernel(nf1, nf2, nf3, nf4, go_embed, go_rad, rel_embed, bn_gamma, bn_beta):
    nf1 = nf1.astype(jnp.int32)
    nf2 = nf2.astype(jnp.int32)
    nf3 = nf3.astype(jnp.int32)
    nf4 = nf4.astype(jnp.int32)
    nf1t, nf2t, nf3t, nf4t = nf1.T, nf2.T, nf3.T, nf4.T
    packed, rad_g = _sc_gather(go_embed, go_rad.reshape(-1),
                               nf1t, nf2t, nf3t, nf4t)
    radsp = rad_g.reshape(NMAT, 2, PB).transpose(0, 2, 1)
    loss = _tc_loss(packed, radsp,
                    nf3t[0].reshape(2, PB).T, nf4t[1].reshape(2, PB).T,
                    rel_embed, go_rad[:16],
                    bn_gamma.reshape(1, D), bn_beta.reshape(1, D))
    return loss[0, 0]
